# baseline (speedup 1.0000x reference)
import functools

import jax
import jax.numpy as jnp
from jax import lax
from jax.experimental import pallas as pl
from jax.experimental.pallas import tpu as pltpu
from jax.experimental.pallas import tpu_sc as plsc

_NUM_BUCKETS = 1000000
_EMBED_DIM = 16
_BATCH = 16384
_NC = 2
_NS = 16
_L = 16
_NW = _NC * _NS
_BPW = _BATCH // _NW
_CH = 128
_NCH = _BPW // _CH


@functools.partial(
    pl.kernel,
    mesh=plsc.VectorSubcoreMesh(core_axis_name="c", subcore_axis_name="s"),
    out_type=jax.ShapeDtypeStruct((_BATCH, _EMBED_DIM), jnp.float32),
    compiler_params=pltpu.CompilerParams(use_tc_tiling_on_sc=False),
    scratch_types=[
        pltpu.VMEM((_BPW,), jnp.int32),
        pltpu.VMEM((_NCH, _CH), jnp.int32),
        pltpu.VMEM((_NCH, _CH), jnp.int32),
        pltpu.VMEM((_BPW, _EMBED_DIM), jnp.float32),
        pltpu.VMEM((_BPW, _EMBED_DIM), jnp.float32),
        pltpu.SemaphoreType.DMA,
    ],
)
def _dh_embed(feat_hbm, table_hbm, out_hbm,
              feat_v, idx1_v, idx2_v, rows1_v, rows2_v, sem):
    wid = lax.axis_index("s") * _NC + lax.axis_index("c")
    base = wid * _BPW
    pltpu.sync_copy(feat_hbm.at[pl.ds(base, _BPW)], feat_v)

    for c in range(_NCH):
        def hash_body(j, _, c=c):
            x = feat_v[pl.ds((c * _CH // _L + j) * _L, _L)].astype(jnp.uint32)
            h1 = (x * jnp.uint32(2654435761)) % jnp.uint32(_NUM_BUCKETS)
            y = x ^ (x >> jnp.uint32(16))
            y = y * jnp.uint32(0x45D9F3B)
            y = y ^ (y >> jnp.uint32(13))
            h2 = y % jnp.uint32(_NUM_BUCKETS)
            idx1_v[c, pl.ds(j * _L, _L)] = h1.astype(jnp.int32)
            idx2_v[c, pl.ds(j * _L, _L)] = h2.astype(jnp.int32)
            return 0

        lax.fori_loop(0, _CH // _L, hash_body, 0)

    copies = []
    for c in range(_NCH):
        copies.append(pltpu.async_copy(
            table_hbm.at[idx1_v.at[c]], rows1_v.at[pl.ds(c * _CH, _CH)], sem))
        copies.append(pltpu.async_copy(
            table_hbm.at[idx2_v.at[c]], rows2_v.at[pl.ds(c * _CH, _CH)], sem))
    for cp in copies:
        cp.wait()

    def add_body(i, _):
        rows1_v[i, :] = rows1_v[i, :] + rows2_v[i, :]
        return 0

    lax.fori_loop(0, _BPW, add_body, 0)

    pltpu.sync_copy(rows1_v, out_hbm.at[pl.ds(base, _BPW)])


def kernel(features, table):
    return _dh_embed(features, table)
